# Initial kernel scaffold; baseline (speedup 1.0000x reference)
import jax
import jax.numpy as jnp
from jax.experimental import pallas as pl


def kernel(input, adj, weight, bias, candidate_weight, update_w, update_b, reset_w, reset_b):
    raise NotImplementedError("write your pallas kernel here")


# speedup vs baseline: 1.0382x; 1.0382x over previous
import functools

import jax
import jax.numpy as jnp
from jax.experimental import pallas as pl

_N = 10000
_D = 128
_BM = 200


def _bdot(a, b):
    return jnp.dot(a.astype(jnp.bfloat16), b.astype(jnp.bfloat16),
                   preferred_element_type=jnp.float32)


def _gru_update(h, m, wu_h, wu_m, wr_h, wr_m, wc, bu, br):
    z = jax.nn.sigmoid(_bdot(h, wu_h) + _bdot(m, wu_m) + bu)
    r = jax.nn.sigmoid(_bdot(h, wr_h) + _bdot(m, wr_m) + br)
    cand = jnp.tanh(_bdot(r * h, wc))
    return z * h + (1.0 - z) * cand


def _h0_kernel(x_ref, w_ref, b_ref, h32_ref, h16_ref):
    h = _bdot(x_ref[...], w_ref[...]) + b_ref[...]
    h32_ref[...] = h
    h16_ref[...] = h.astype(jnp.bfloat16)


def _step_kernel(adj_ref, hb_ref, h_ref, wu_h_ref, wu_m_ref, wr_h_ref,
                 wr_m_ref, wc_ref, bu_ref, br_ref, nh32_ref, nh16_ref,
                 *maybe_adj16_ref, cast_adj):
    adj = adj_ref[...].astype(jnp.bfloat16)
    if cast_adj:
        maybe_adj16_ref[0][...] = adj
    m = jnp.dot(adj, hb_ref[...], preferred_element_type=jnp.float32)
    h_new = _gru_update(h_ref[...], m, wu_h_ref[...], wu_m_ref[...],
                        wr_h_ref[...], wr_m_ref[...], wc_ref[...],
                        bu_ref[...], br_ref[...])
    nh32_ref[...] = h_new
    nh16_ref[...] = h_new.astype(jnp.bfloat16)


def _row_spec(bm, width):
    return pl.BlockSpec((bm, width), lambda i: (i, 0))


def _full_spec(shape):
    return pl.BlockSpec(shape, lambda i: (0, 0))


def kernel(input, adj, weight, bias, candidate_weight, update_w, update_b,
           reset_w, reset_b):
    wu_h, wu_m = update_w[:_D], update_w[_D:]
    wr_h, wr_m = reset_w[:_D], reset_w[_D:]
    bu = update_b.reshape(1, _D)
    br = reset_b.reshape(1, _D)

    h_shapes = [jax.ShapeDtypeStruct((_N, _D), jnp.float32),
                jax.ShapeDtypeStruct((_N, _D), jnp.bfloat16)]

    h32, h16 = pl.pallas_call(
        _h0_kernel,
        grid=(_N // 1000,),
        in_specs=[_row_spec(1000, _D), _full_spec((_D, _D)),
                  _full_spec((1, _D))],
        out_specs=[_row_spec(1000, _D), _row_spec(1000, _D)],
        out_shape=h_shapes,
    )(input, weight, bias.reshape(1, _D))

    small_specs = [
        _full_spec((_D, _D)), _full_spec((_D, _D)), _full_spec((_D, _D)),
        _full_spec((_D, _D)), _full_spec((_D, _D)), _full_spec((1, _D)),
        _full_spec((1, _D)),
    ]
    small_args = (wu_h, wu_m, wr_h, wr_m, candidate_weight, bu, br)
    h_out_specs = [_row_spec(_BM, _D), _row_spec(_BM, _D)]

    h32, h16, adj16 = pl.pallas_call(
        functools.partial(_step_kernel, cast_adj=True),
        grid=(_N // _BM,),
        in_specs=[_row_spec(_BM, _N), _full_spec((_N, _D)),
                  _row_spec(_BM, _D)] + small_specs,
        out_specs=h_out_specs + [_row_spec(_BM, _N)],
        out_shape=h_shapes + [jax.ShapeDtypeStruct((_N, _N), jnp.bfloat16)],
    )(adj, h16, h32, *small_args)

    step = pl.pallas_call(
        functools.partial(_step_kernel, cast_adj=False),
        grid=(_N // _BM,),
        in_specs=[_row_spec(_BM, _N), _full_spec((_N, _D)),
                  _row_spec(_BM, _D)] + small_specs,
        out_specs=h_out_specs,
        out_shape=h_shapes,
    )
    h32, h16 = step(adj16, h16, h32, *small_args)
    h32, _ = step(adj16, h16, h32, *small_args)
    return h32
